# Initial kernel scaffold; baseline (speedup 1.0000x reference)
import jax
import jax.numpy as jnp
from jax.experimental import pallas as pl


def kernel(x, edge_index, W_l, b_l, W_r):
    raise NotImplementedError("write your pallas kernel here")


# speedup vs baseline: 8.8598x; 8.8598x over previous
import functools
import jax
import jax.numpy as jnp
from jax import lax
from jax.experimental import pallas as pl
from jax.experimental.pallas import tpu as pltpu
from jax.experimental.pallas import tpu_sc as plsc

N_NODES = 10000
E_EDGES = 320000
DIM = 128

NUM_CORES = 2
NUM_SUBCORES = 16
NUM_WORKERS = NUM_CORES * NUM_SUBCORES
EDGES_PER_W = E_EDGES // NUM_WORKERS
CHUNK = 125
NCHUNK = EDGES_PER_W // CHUNK
N_PAD = 10240
ROWS_PER_SUB = N_PAD // NUM_SUBCORES
ZROWS = 128


def _sc_aggregate(x, src_r, dst_r):
    mesh = plsc.VectorSubcoreMesh(core_axis_name="c", subcore_axis_name="s")

    @functools.partial(
        pl.kernel,
        mesh=mesh,
        out_type=jax.ShapeDtypeStruct((NUM_CORES, N_PAD, DIM), jnp.float32),
        scratch_types=[
            pltpu.VMEM((NCHUNK, CHUNK), jnp.int32),
            pltpu.VMEM((NCHUNK, CHUNK), jnp.int32),
            pltpu.VMEM((ZROWS, DIM), jnp.float32),
            pltpu.VMEM_SHARED((N_PAD, DIM), jnp.float32),
            pltpu.SemaphoreType.DMA,
        ],
    )
    def sc_kernel(x_hbm, src_hbm, dst_hbm, out_hbm,
                  src_v, dst_v, rows_v, aggr_sh, gsem):
        c = lax.axis_index("c")
        s = lax.axis_index("s")
        wid = c * NUM_SUBCORES + s

        pltpu.sync_copy(src_hbm.at[wid], src_v)
        pltpu.sync_copy(dst_hbm.at[wid], dst_v)

        zeros16 = jnp.zeros((16,), jnp.float32)

        def zbody(i, carry):
            rows_v[i // 8, pl.ds((i % 8) * 16, 16)] = zeros16
            return carry

        lax.fori_loop(0, ZROWS * 8, zbody, 0, unroll=8)

        for r in range(ROWS_PER_SUB // ZROWS):
            pltpu.sync_copy(
                rows_v, aggr_sh.at[pl.ds(s * ROWS_PER_SUB + r * ZROWS, ZROWS)]
            )
        plsc.subcore_barrier()

        def body(j, carry):
            chunk = rows_v.at[pl.ds(0, CHUNK)]
            pltpu.async_copy(x_hbm.at[src_v.at[j]], chunk, gsem).wait()
            pltpu.sync_copy(chunk, aggr_sh.at[dst_v.at[j]], add=True)
            return carry

        lax.fori_loop(0, NCHUNK, body, 0)
        plsc.subcore_barrier()

        pltpu.sync_copy(
            aggr_sh.at[pl.ds(s * ROWS_PER_SUB, ROWS_PER_SUB)],
            out_hbm.at[c, pl.ds(s * ROWS_PER_SUB, ROWS_PER_SUB)],
        )

    return sc_kernel(x, src_r, dst_r)


def _tc_tail(partials, x, W_l, b_l, W_r):
    def tc_kernel(p_ref, x_ref, wl_ref, wr_ref, bl_ref, o_ref):
        aggr = p_ref[0, :N_NODES, :] + p_ref[1, :N_NODES, :]
        h = lax.dot_general(
            aggr, wl_ref[...], (((1,), (1,)), ((), ())),
            preferred_element_type=jnp.float32,
        )
        h = h + lax.dot_general(
            x_ref[...], wr_ref[...], (((1,), (1,)), ((), ())),
            preferred_element_type=jnp.float32,
        )
        o_ref[...] = jnp.maximum(h + bl_ref[...], 0.0)

    return pl.pallas_call(
        tc_kernel,
        out_shape=jax.ShapeDtypeStruct((N_NODES, DIM), jnp.float32),
    )(partials, x, W_l, W_r, b_l.reshape(1, DIM))


@jax.jit
def kernel(x, edge_index, W_l, b_l, W_r):
    src_r = edge_index[0].reshape(NUM_WORKERS, NCHUNK, CHUNK)
    dst_r = edge_index[1].reshape(NUM_WORKERS, NCHUNK, CHUNK)
    partials = _sc_aggregate(x, src_r, dst_r)
    return _tc_tail(partials, x, W_l, b_l, W_r)


# speedup vs baseline: 12.3692x; 1.3961x over previous
import functools
import jax
import jax.numpy as jnp
from jax import lax
from jax.experimental import pallas as pl
from jax.experimental.pallas import tpu as pltpu
from jax.experimental.pallas import tpu_sc as plsc

N_NODES = 10000
E_EDGES = 320000
DIM = 128

NUM_CORES = 2
NUM_SUBCORES = 16
NUM_WORKERS = NUM_CORES * NUM_SUBCORES
EDGES_PER_W = E_EDGES // NUM_WORKERS
CHUNK = 80
NCHUNK = EDGES_PER_W // CHUNK
N_PAD = 10240
ROWS_PER_SUB = N_PAD // NUM_SUBCORES


def _sc_aggregate(x, src_r, dst_r):
    mesh = plsc.VectorSubcoreMesh(core_axis_name="c", subcore_axis_name="s")

    @functools.partial(
        pl.kernel,
        mesh=mesh,
        out_type=jax.ShapeDtypeStruct((NUM_CORES, N_PAD, DIM), jnp.float32),
        scratch_types=[
            pltpu.VMEM((EDGES_PER_W,), jnp.int32),
            pltpu.VMEM((NCHUNK, CHUNK), jnp.int32),
            pltpu.VMEM((CHUNK, DIM), jnp.float32),
            pltpu.VMEM((CHUNK, DIM), jnp.float32),
            pltpu.VMEM_SHARED((N_PAD, DIM), jnp.float32),
            pltpu.SemaphoreType.DMA,
            pltpu.SemaphoreType.DMA,
        ],
    )
    def sc_kernel(x_hbm, src_hbm, dst_hbm, out_hbm,
                  src_v, dst_v, rows0, rows1, aggr_sh, sem0, sem1):
        c = lax.axis_index("c")
        s = lax.axis_index("s")
        wid = c * NUM_SUBCORES + s

        idx_cp0 = pltpu.async_copy(src_hbm.at[wid], src_v, sem0)
        idx_cp1 = pltpu.async_copy(dst_hbm.at[wid], dst_v, sem1)

        zeros16 = jnp.zeros((16,), jnp.float32)

        def zbody(i, carry):
            rows0[i // 8, pl.ds((i % 8) * 16, 16)] = zeros16
            return carry

        lax.fori_loop(0, CHUNK * 8, zbody, 0, unroll=8)

        base = s * ROWS_PER_SUB
        for r in range(ROWS_PER_SUB // CHUNK):
            pltpu.sync_copy(rows0,
                            aggr_sh.at[pl.ds(base + r * CHUNK, CHUNK)])
        idx_cp0.wait()
        idx_cp1.wait()
        plsc.subcore_barrier()

        def gref(j):
            return x_hbm.at[src_v.at[pl.ds(j * CHUNK, CHUNK)]]

        pltpu.async_copy(gref(0), rows0, sem0)

        def body(i, carry):
            j = 2 * i
            pltpu.async_copy(gref(j + 1), rows1, sem1)
            pltpu.make_async_copy(gref(j), rows0, sem0).wait()
            pltpu.sync_copy(rows0, aggr_sh.at[dst_v.at[j]], add=True)
            pltpu.async_copy(gref(j + 2), rows0, sem0)
            pltpu.make_async_copy(gref(j + 1), rows1, sem1).wait()
            pltpu.sync_copy(rows1, aggr_sh.at[dst_v.at[j + 1]], add=True)
            return carry

        lax.fori_loop(0, (NCHUNK - 1) // 2, body, 0)

        pltpu.make_async_copy(gref(NCHUNK - 1), rows0, sem0).wait()
        pltpu.sync_copy(rows0, aggr_sh.at[dst_v.at[NCHUNK - 1]], add=True)
        plsc.subcore_barrier()

        pltpu.sync_copy(
            aggr_sh.at[pl.ds(base, ROWS_PER_SUB)],
            out_hbm.at[c, pl.ds(base, ROWS_PER_SUB)],
        )

    return sc_kernel(x, src_r, dst_r)


def _tc_tail(partials, x, W_l, b_l, W_r):
    def tc_kernel(p_ref, x_ref, wl_ref, wr_ref, bl_ref, o_ref):
        aggr = p_ref[0, :N_NODES, :] + p_ref[1, :N_NODES, :]
        h = lax.dot_general(
            aggr, wl_ref[...], (((1,), (1,)), ((), ())),
            preferred_element_type=jnp.float32,
        )
        h = h + lax.dot_general(
            x_ref[...], wr_ref[...], (((1,), (1,)), ((), ())),
            preferred_element_type=jnp.float32,
        )
        o_ref[...] = jnp.maximum(h + bl_ref[...], 0.0)

    return pl.pallas_call(
        tc_kernel,
        out_shape=jax.ShapeDtypeStruct((N_NODES, DIM), jnp.float32),
    )(partials, x, W_l, W_r, b_l.reshape(1, DIM))


@jax.jit
def kernel(x, edge_index, W_l, b_l, W_r):
    src_r = edge_index[0].reshape(NUM_WORKERS, EDGES_PER_W)
    dst_r = edge_index[1].reshape(NUM_WORKERS, NCHUNK, CHUNK)
    partials = _sc_aggregate(x, src_r, dst_r)
    return _tc_tail(partials, x, W_l, b_l, W_r)
